# Initial kernel scaffold; baseline (speedup 1.0000x reference)
import jax
import jax.numpy as jnp
from jax.experimental import pallas as pl


def kernel(x, edge_index, edge_attr, W_elec, b_elec, W_chem, b_chem, W_lin, b_lin):
    raise NotImplementedError("write your pallas kernel here")


# speedup vs baseline: 10.1536x; 10.1536x over previous
import jax
import jax.numpy as jnp
from jax import lax
from jax.experimental import pallas as pl
from jax.experimental.pallas import tpu as pltpu
from jax.experimental.pallas import tpu_sc as plsc

N = 10000
D = 128
DH2 = 256
E_IN = 320000
E_TOT = E_IN + N

NC, NS, L = 2, 16, 16
NTILES = NC * NS

W = 64
WPT = -(-E_TOT // (NTILES * W))
E_PAD = NTILES * WPT * W
E_ROWS = E_PAD // W

NPAD = 10240
RPT = NPAD // NS
DROWS_PT = E_ROWS // NS
DCH = 9
MAGIC = 0x5F3759DF


def _sc_body(src2d, dst2d, w1_2d, w2_2d, h_hbm, acc_hbm,
             dinv1_t, dinv2_t, rows_t, msg_t, srcv, dstv, wv1, wv2,
             dstd, wd1, wd2, nbuf1, nbuf2,
             deg1_sh, deg2_sh, acc_sh):
    cid = lax.axis_index("c")
    sid = lax.axis_index("s")
    zeros16 = jnp.zeros((L,), jnp.float32)

    @pl.loop(0, W)
    def _(r):
        for c in range(D // L):
            msg_t[r, pl.ds(c * L, L)] = zeros16

    @pl.loop(0, RPT // W)
    def _(k):
        pltpu.sync_copy(msg_t, acc_sh.at[pl.ds(sid * RPT + k * W, W)])

    @pl.loop(0, DCH)
    def _(j):
        for c in range(W // L):
            wd1[j, pl.ds(c * L, L)] = zeros16

    @pl.loop(0, RPT // W)
    def _(k):
        pltpu.sync_copy(wd1.at[0], deg1_sh.at[pl.ds(sid * RPT + k * W, W)])
        pltpu.sync_copy(wd1.at[0], deg2_sh.at[pl.ds(sid * RPT + k * W, W)])

    plsc.subcore_barrier()

    @pl.loop(0, DROWS_PT // DCH)
    def _(chunk):
        row0 = sid * DROWS_PT + chunk * DCH
        pltpu.sync_copy(dst2d.at[pl.ds(row0, DCH)], dstd)
        pltpu.sync_copy(w1_2d.at[pl.ds(row0, DCH)], wd1)
        pltpu.sync_copy(w2_2d.at[pl.ds(row0, DCH)], wd2)
        for j in range(DCH):
            pltpu.sync_copy(wd1.at[j], deg1_sh.at[dstd.at[j]], add=True)
            pltpu.sync_copy(wd2.at[j], deg2_sh.at[dstd.at[j]], add=True)

    plsc.subcore_barrier()

    off = sid * RPT

    @pl.loop(0, RPT // W)
    def _(k):
        pltpu.sync_copy(deg1_sh.at[pl.ds(off + k * W, W)], wd1.at[0])
        pltpu.sync_copy(deg2_sh.at[pl.ds(off + k * W, W)], wd2.at[0])
        for j, buf in ((0, wd1), (1, wd2)):
            for g in range(W // L):
                d = buf[0, pl.ds(g * L, L)]
                bits = plsc.bitcast(d, jnp.int32)
                y = plsc.bitcast(jnp.int32(MAGIC) - (bits >> 1), jnp.float32)
                for _ in range(3):
                    y = y * (1.5 - 0.5 * d * y * y)
                buf[0, pl.ds(g * L, L)] = y
        pltpu.sync_copy(wd1.at[0], deg1_sh.at[pl.ds(off + k * W, W)])
        pltpu.sync_copy(wd2.at[0], deg2_sh.at[pl.ds(off + k * W, W)])

    plsc.subcore_barrier()
    pltpu.sync_copy(deg1_sh, dinv1_t)
    pltpu.sync_copy(deg2_sh, dinv2_t)

    wbase = cid * (E_ROWS // NC) + sid * WPT

    @pl.loop(0, WPT)
    def _(k):
        row = wbase + k
        pltpu.sync_copy(src2d.at[pl.ds(row, 1)], srcv)
        pltpu.sync_copy(dst2d.at[pl.ds(row, 1)], dstv)
        pltpu.sync_copy(w1_2d.at[pl.ds(row, 1)], wv1)
        pltpu.sync_copy(w2_2d.at[pl.ds(row, 1)], wv2)
        pltpu.sync_copy(h_hbm.at[srcv.at[0]], rows_t)
        for g in range(W // L):
            s16 = srcv[0, pl.ds(g * L, L)]
            d16 = dstv[0, pl.ds(g * L, L)]
            n1 = (plsc.load_gather(dinv1_t, [s16]) * wv1[0, pl.ds(g * L, L)]
                  * plsc.load_gather(dinv1_t, [d16]))
            n2 = (plsc.load_gather(dinv2_t, [s16]) * wv2[0, pl.ds(g * L, L)]
                  * plsc.load_gather(dinv2_t, [d16]))
            nbuf1[pl.ds(g * L, L)] = n1
            nbuf2[pl.ds(g * L, L)] = n2

        @pl.loop(0, W)
        def _(e):
            eidx = jnp.full((L,), e, jnp.int32)
            bn1 = plsc.load_gather(nbuf1, [eidx])
            bn2 = plsc.load_gather(nbuf2, [eidx])
            for c in range(D // L):
                r1 = rows_t[e, pl.ds(c * L, L)]
                r2 = rows_t[e, pl.ds(D + c * L, L)]
                msg_t[e, pl.ds(c * L, L)] = bn1 * r1 + bn2 * r2
        pltpu.sync_copy(msg_t, acc_sh.at[dstv.at[0]], add=True)

    plsc.subcore_barrier()
    obase = cid * NPAD + sid * RPT
    pltpu.sync_copy(acc_sh.at[pl.ds(sid * RPT, RPT)], acc_hbm.at[pl.ds(obase, RPT)])


def _mm_body(x_ref, we_ref, wc_ref, wl_ref, h_ref):
    wl = wl_ref[...]
    m = jnp.concatenate(
        [jnp.dot(we_ref[...], wl[:D], preferred_element_type=jnp.float32),
         jnp.dot(wc_ref[...], wl[D:], preferred_element_type=jnp.float32)],
        axis=1)
    h_ref[...] = jnp.dot(x_ref[...], m, preferred_element_type=jnp.float32)


def _fin_body(a0_ref, a1_ref, be_ref, bc_ref, wl_ref, bl_ref, o_ref):
    wl = wl_ref[...]
    crow = (jnp.dot(be_ref[...], wl[:D], preferred_element_type=jnp.float32)
            + jnp.dot(bc_ref[...], wl[D:], preferred_element_type=jnp.float32)
            + bl_ref[...])
    o_ref[...] = a0_ref[0] + a1_ref[0] + crow


def kernel(x, edge_index, edge_attr, W_elec, b_elec, W_chem, b_chem, W_lin, b_lin):
    src = edge_index[0].astype(jnp.int32)
    dst = edge_index[1].astype(jnp.int32)
    loop = jnp.arange(N, dtype=jnp.int32)
    padi = jnp.zeros((E_PAD - E_TOT,), jnp.int32)
    padf = jnp.zeros((E_PAD - E_TOT,), jnp.float32)
    two = jnp.full((N,), 2.0, jnp.float32)
    src2d = jnp.concatenate([src, loop, padi]).reshape(E_ROWS, W)
    dst2d = jnp.concatenate([dst, loop, padi]).reshape(E_ROWS, W)
    w1_2d = jnp.concatenate([edge_attr[:, 0], two, padf]).reshape(E_ROWS, W)
    w2_2d = jnp.concatenate([edge_attr[:, 1], two, padf]).reshape(E_ROWS, W)

    nblk = 10
    h = pl.pallas_call(
        _mm_body,
        grid=(nblk,),
        in_specs=[
            pl.BlockSpec((N // nblk, D), lambda i: (i, 0)),
            pl.BlockSpec((D, D), lambda i: (0, 0)),
            pl.BlockSpec((D, D), lambda i: (0, 0)),
            pl.BlockSpec((2 * D, D), lambda i: (0, 0)),
        ],
        out_specs=pl.BlockSpec((N // nblk, DH2), lambda i: (i, 0)),
        out_shape=jax.ShapeDtypeStruct((N, DH2), jnp.float32),
    )(x, W_elec, W_chem, W_lin)

    mesh = plsc.VectorSubcoreMesh(core_axis_name="c", subcore_axis_name="s",
                                  num_cores=NC, num_subcores=NS)
    sc_fn = pl.kernel(
        _sc_body,
        out_type=jax.ShapeDtypeStruct((NC * NPAD, D), jnp.float32),
        mesh=mesh,
        compiler_params=pltpu.CompilerParams(use_tc_tiling_on_sc=False,
                                             needs_layout_passes=False),
        scratch_types=[
            pltpu.VMEM((NPAD,), jnp.float32),
            pltpu.VMEM((NPAD,), jnp.float32),
            pltpu.VMEM((W, DH2), jnp.float32),
            pltpu.VMEM((W, D), jnp.float32),
            pltpu.VMEM((1, W), jnp.int32),
            pltpu.VMEM((1, W), jnp.int32),
            pltpu.VMEM((1, W), jnp.float32),
            pltpu.VMEM((1, W), jnp.float32),
            pltpu.VMEM((DCH, W), jnp.int32),
            pltpu.VMEM((DCH, W), jnp.float32),
            pltpu.VMEM((DCH, W), jnp.float32),
            pltpu.VMEM((W,), jnp.float32),
            pltpu.VMEM((W,), jnp.float32),
            pltpu.VMEM_SHARED((NPAD,), jnp.float32),
            pltpu.VMEM_SHARED((NPAD,), jnp.float32),
            pltpu.VMEM_SHARED((NPAD, D), jnp.float32),
        ],
    )
    acc = sc_fn(src2d, dst2d, w1_2d, w2_2d, h)
    acc3d = acc.reshape(NC, NPAD, D)

    out = pl.pallas_call(
        _fin_body,
        grid=(nblk,),
        in_specs=[
            pl.BlockSpec((1, N // nblk, D), lambda i: (0, i, 0)),
            pl.BlockSpec((1, N // nblk, D), lambda i: (1, i, 0)),
            pl.BlockSpec((1, D), lambda i: (0, 0)),
            pl.BlockSpec((1, D), lambda i: (0, 0)),
            pl.BlockSpec((2 * D, D), lambda i: (0, 0)),
            pl.BlockSpec((1, D), lambda i: (0, 0)),
        ],
        out_specs=pl.BlockSpec((N // nblk, D), lambda i: (i, 0)),
        out_shape=jax.ShapeDtypeStruct((N, D), jnp.float32),
    )(acc3d, acc3d, b_elec.reshape(1, D), b_chem.reshape(1, D),
      W_lin, b_lin.reshape(1, D))
    return out


# speedup vs baseline: 12.7210x; 1.2528x over previous
import jax
import jax.numpy as jnp
from jax import lax
from jax.experimental import pallas as pl
from jax.experimental.pallas import tpu as pltpu
from jax.experimental.pallas import tpu_sc as plsc

N = 10000
D = 128
DH2 = 256
E_IN = 320000
E_TOT = E_IN + N

NC, NS, L = 2, 16, 16
NTILES = NC * NS

WS = 32
SPT = 2 * (-(-E_TOT // (NTILES * WS * 2)))
E_PAD = NTILES * SPT * WS
E32 = E_PAD // WS
E128 = E_PAD // 128
DROWS_PT = E128 // NS
DCH = 6
assert SPT % 2 == 0 and E128 * 128 == E_PAD
assert DROWS_PT * NS == E128 and DROWS_PT % DCH == 0

NPAD = 10240
RPT = NPAD // NS
MAGIC = 0x5F3759DF


def _sc_body(ei3, wv3, dstE, w1E, w2E, h_hbm, acc_hbm,
             dinv1_t, dinv2_t, rows2, msg2, eb, db, wb,
             dstd, wd1, wd2,
             gsem0, gsem1, ssem0, ssem1, dsem,
             deg1_sh, deg2_sh, acc_sh):
    cid = lax.axis_index("c")
    sid = lax.axis_index("s")
    zeros16 = jnp.zeros((L,), jnp.float32)
    gsem = (gsem0, gsem1)
    ssem = (ssem0, ssem1)

    @pl.loop(0, WS)
    def _(r):
        for c in range(D // L):
            msg2[0, r, pl.ds(c * L, L)] = zeros16
            msg2[1, r, pl.ds(c * L, L)] = zeros16

    @pl.loop(0, DCH)
    def _(j):
        for c in range(128 // L):
            wd1[j, pl.ds(c * L, L)] = zeros16

    pend = []
    for k in range(RPT // (2 * WS)):
        for b in range(2):
            pend.append(pltpu.async_copy(
                msg2.at[b], acc_sh.at[pl.ds(sid * RPT + (2 * k + b) * WS, WS)], dsem))
    for k in range(RPT // 128):
        pend.append(pltpu.async_copy(
            wd1.at[0], deg1_sh.at[pl.ds(sid * RPT + k * 128, 128)], dsem))
        pend.append(pltpu.async_copy(
            wd1.at[0], deg2_sh.at[pl.ds(sid * RPT + k * 128, 128)], dsem))
    for p in pend:
        p.wait()
    plsc.subcore_barrier()

    @pl.loop(0, DROWS_PT // DCH)
    def _(chunk):
        row0 = sid * DROWS_PT + chunk * DCH
        pltpu.sync_copy(dstE.at[pl.ds(row0, DCH)], dstd)
        pltpu.sync_copy(w1E.at[pl.ds(row0, DCH)], wd1)
        pltpu.sync_copy(w2E.at[pl.ds(row0, DCH)], wd2)
        pend2 = []
        for j in range(DCH):
            pend2.append(pltpu.async_copy(
                wd1.at[j], deg1_sh.at[dstd.at[j]], dsem, add=True))
            pend2.append(pltpu.async_copy(
                wd2.at[j], deg2_sh.at[dstd.at[j]], dsem, add=True))
        for p in pend2:
            p.wait()

    plsc.subcore_barrier()

    off = sid * RPT

    @pl.loop(0, RPT // 128)
    def _(k):
        pltpu.sync_copy(deg1_sh.at[pl.ds(off + k * 128, 128)], wd1.at[0])
        pltpu.sync_copy(deg2_sh.at[pl.ds(off + k * 128, 128)], wd2.at[0])
        for buf in (wd1, wd2):
            for g in range(128 // L):
                d = buf[0, pl.ds(g * L, L)]
                bits = plsc.bitcast(d, jnp.int32)
                y = plsc.bitcast(jnp.int32(MAGIC) - (bits >> 1), jnp.float32)
                for _ in range(3):
                    y = y * (1.5 - 0.5 * d * y * y)
                buf[0, pl.ds(g * L, L)] = y
        pltpu.sync_copy(wd1.at[0], deg1_sh.at[pl.ds(off + k * 128, 128)])
        pltpu.sync_copy(wd2.at[0], deg2_sh.at[pl.ds(off + k * 128, 128)])

    plsc.subcore_barrier()
    pltpu.sync_copy(deg1_sh.at[pl.ds(0, N)], dinv1_t)
    pltpu.sync_copy(deg2_sh.at[pl.ds(0, N)], dinv2_t)

    sbase = cid * (E32 // NC) + sid * SPT

    def load_step(k, b):
        pltpu.sync_copy(ei3.at[pl.ds(sbase + k, 1)], eb.at[b])
        pltpu.sync_copy(wv3.at[pl.ds(sbase + k, 1)], wb.at[b])

    def issue_gather(b):
        return pltpu.async_copy(h_hbm.at[eb.at[b, 0, 0]], rows2.at[b], gsem[b])

    def wait_gather(b):
        pltpu.make_async_copy(h_hbm.at[eb.at[b, 0, 0]], rows2.at[b], gsem[b]).wait()

    def issue_scatter(b):
        return pltpu.async_copy(msg2.at[b], acc_sh.at[db.at[b, 0]], ssem[b],
                                add=True)

    def wait_scatter(b):
        pltpu.make_async_copy(msg2.at[b], acc_sh.at[db.at[b, 0]], ssem[b]).wait()

    load_step(0, 0)
    load_step(1, 1)
    issue_gather(0)
    issue_gather(1)

    def step_half(k, b, first):
        if not first:
            wait_scatter(b)
        for g in range(WS // L):
            s16 = eb[b, 0, 0, pl.ds(g * L, L)]
            d16 = eb[b, 0, 1, pl.ds(g * L, L)]
            db[b, 0, pl.ds(g * L, L)] = d16
            n1 = (plsc.load_gather(dinv1_t, [s16]) * wb[b, 0, 0, pl.ds(g * L, L)]
                  * plsc.load_gather(dinv1_t, [d16]))
            n2 = (plsc.load_gather(dinv2_t, [s16]) * wb[b, 0, 1, pl.ds(g * L, L)]
                  * plsc.load_gather(dinv2_t, [d16]))
            wb[b, 0, 0, pl.ds(g * L, L)] = n1
            wb[b, 0, 1, pl.ds(g * L, L)] = n2
        wait_gather(b)
        @pl.loop(0, WS)
        def _(e):
            eidx = jnp.full((L,), e, jnp.int32)
            bn1 = plsc.load_gather(wb.at[b, 0, 0], [eidx])
            bn2 = plsc.load_gather(wb.at[b, 0, 1], [eidx])
            for c in range(D // L):
                r1 = rows2[b, e, pl.ds(c * L, L)]
                r2 = rows2[b, e, pl.ds(D + c * L, L)]
                msg2[b, e, pl.ds(c * L, L)] = bn1 * r1 + bn2 * r2
        issue_scatter(b)
        load_step(k + 2, b)
        issue_gather(b)

    step_half(0, 0, True)
    step_half(1, 1, True)

    @pl.loop(2, SPT - 2, step=2)
    def _(k):
        step_half(k, 0, False)
        step_half(k + 1, 1, False)

    step_half(SPT - 2, 0, False)
    step_half(SPT - 1, 1, False)
    wait_gather(0)
    wait_gather(1)
    wait_scatter(0)
    wait_scatter(1)

    plsc.subcore_barrier()
    obase = cid * NPAD + sid * RPT
    pltpu.sync_copy(acc_sh.at[pl.ds(sid * RPT, RPT)], acc_hbm.at[pl.ds(obase, RPT)])


def _mm_body(x_ref, we_ref, wc_ref, wl_ref, h_ref):
    wl = wl_ref[...]
    m = jnp.concatenate(
        [jnp.dot(we_ref[...], wl[:D], preferred_element_type=jnp.float32),
         jnp.dot(wc_ref[...], wl[D:], preferred_element_type=jnp.float32)],
        axis=1)
    h_ref[...] = jnp.dot(x_ref[...], m, preferred_element_type=jnp.float32)


def _fin_body(a0_ref, a1_ref, be_ref, bc_ref, wl_ref, bl_ref, o_ref):
    wl = wl_ref[...]
    crow = (jnp.dot(be_ref[...], wl[:D], preferred_element_type=jnp.float32)
            + jnp.dot(bc_ref[...], wl[D:], preferred_element_type=jnp.float32)
            + bl_ref[...])
    o_ref[...] = a0_ref[0] + a1_ref[0] + crow


def kernel(x, edge_index, edge_attr, W_elec, b_elec, W_chem, b_chem, W_lin, b_lin):
    src = edge_index[0].astype(jnp.int32)
    dst = edge_index[1].astype(jnp.int32)
    loop = jnp.arange(N, dtype=jnp.int32)
    padi = jnp.zeros((E_PAD - E_TOT,), jnp.int32)
    padf = jnp.zeros((E_PAD - E_TOT,), jnp.float32)
    two = jnp.full((N,), 2.0, jnp.float32)
    srcp = jnp.concatenate([src, loop, padi])
    dstp = jnp.concatenate([dst, loop, padi])
    w1p = jnp.concatenate([edge_attr[:, 0], two, padf])
    w2p = jnp.concatenate([edge_attr[:, 1], two, padf])
    ei3 = jnp.concatenate(
        [jnp.stack([srcp.reshape(E32, WS), dstp.reshape(E32, WS)], axis=1),
         jnp.zeros((2, 2, WS), jnp.int32)])
    wv3 = jnp.concatenate(
        [jnp.stack([w1p.reshape(E32, WS), w2p.reshape(E32, WS)], axis=1),
         jnp.zeros((2, 2, WS), jnp.float32)])
    dstE = dstp.reshape(E128, 128)
    w1E = w1p.reshape(E128, 128)
    w2E = w2p.reshape(E128, 128)

    nblk = 10
    h = pl.pallas_call(
        _mm_body,
        grid=(nblk,),
        in_specs=[
            pl.BlockSpec((N // nblk, D), lambda i: (i, 0)),
            pl.BlockSpec((D, D), lambda i: (0, 0)),
            pl.BlockSpec((D, D), lambda i: (0, 0)),
            pl.BlockSpec((2 * D, D), lambda i: (0, 0)),
        ],
        out_specs=pl.BlockSpec((N // nblk, DH2), lambda i: (i, 0)),
        out_shape=jax.ShapeDtypeStruct((N, DH2), jnp.float32),
    )(x, W_elec, W_chem, W_lin)

    mesh = plsc.VectorSubcoreMesh(core_axis_name="c", subcore_axis_name="s",
                                  num_cores=NC, num_subcores=NS)
    sc_fn = pl.kernel(
        _sc_body,
        out_type=jax.ShapeDtypeStruct((NC * NPAD, D), jnp.float32),
        mesh=mesh,
        compiler_params=pltpu.CompilerParams(use_tc_tiling_on_sc=False,
                                             needs_layout_passes=False),
        scratch_types=[
            pltpu.VMEM((N,), jnp.float32),
            pltpu.VMEM((N,), jnp.float32),
            pltpu.VMEM((2, WS, DH2), jnp.float32),
            pltpu.VMEM((2, WS, D), jnp.float32),
            pltpu.VMEM((2, 1, 2, WS), jnp.int32),
            pltpu.VMEM((2, 1, WS), jnp.int32),
            pltpu.VMEM((2, 1, 2, WS), jnp.float32),
            pltpu.VMEM((DCH, 128), jnp.int32),
            pltpu.VMEM((DCH, 128), jnp.float32),
            pltpu.VMEM((DCH, 128), jnp.float32),
            pltpu.SemaphoreType.DMA,
            pltpu.SemaphoreType.DMA,
            pltpu.SemaphoreType.DMA,
            pltpu.SemaphoreType.DMA,
            pltpu.SemaphoreType.DMA,
            pltpu.VMEM_SHARED((NPAD,), jnp.float32),
            pltpu.VMEM_SHARED((NPAD,), jnp.float32),
            pltpu.VMEM_SHARED((NPAD, D), jnp.float32),
        ],
    )
    acc = sc_fn(ei3, wv3, dstE, w1E, w2E, h)
    acc3d = acc.reshape(NC, NPAD, D)

    out = pl.pallas_call(
        _fin_body,
        grid=(nblk,),
        in_specs=[
            pl.BlockSpec((1, N // nblk, D), lambda i: (0, i, 0)),
            pl.BlockSpec((1, N // nblk, D), lambda i: (1, i, 0)),
            pl.BlockSpec((1, D), lambda i: (0, 0)),
            pl.BlockSpec((1, D), lambda i: (0, 0)),
            pl.BlockSpec((2 * D, D), lambda i: (0, 0)),
            pl.BlockSpec((1, D), lambda i: (0, 0)),
        ],
        out_specs=pl.BlockSpec((N // nblk, D), lambda i: (i, 0)),
        out_shape=jax.ShapeDtypeStruct((N, D), jnp.float32),
    )(acc3d, acc3d, b_elec.reshape(1, D), b_chem.reshape(1, D),
      W_lin, b_lin.reshape(1, D))
    return out


# speedup vs baseline: 17.0244x; 1.3383x over previous
import jax
import jax.numpy as jnp
from jax import lax
from jax.experimental import pallas as pl
from jax.experimental.pallas import tpu as pltpu
from jax.experimental.pallas import tpu_sc as plsc

N = 10000
D = 128
DH2 = 256
E_IN = 320000
E_TOT = E_IN + N

NC, NS, L = 2, 16, 16
NTILES = NC * NS

WS = 32
SPT = 2 * (-(-E_TOT // (NTILES * WS * 2)))
E_PAD = NTILES * SPT * WS
E32 = E_PAD // WS
E128 = E_PAD // 128
DROWS_PT = E128 // NS
DCH = 6
assert SPT % 2 == 0 and E128 * 128 == E_PAD
assert DROWS_PT * NS == E128 and DROWS_PT % DCH == 0

NPAD = 10240
RPT = NPAD // NS
MAGIC = 0x5F3759DF


def _sc_body(exy, dstE, w1E, w2E, h_hbm, acc_hbm,
             dinv1_t, dinv2_t, rows2, msg2, eb, db, nb,
             dstd, wd1, wd2,
             gsem0, gsem1, ssem0, ssem1, esem0, esem1, dsem,
             deg1_sh, deg2_sh, acc_sh):
    cid = lax.axis_index("c")
    sid = lax.axis_index("s")
    zeros16 = jnp.zeros((L,), jnp.float32)
    gsem = (gsem0, gsem1)
    ssem = (ssem0, ssem1)
    esem = (esem0, esem1)

    @pl.loop(0, WS)
    def _(r):
        for c in range(D // L):
            msg2[0, r, pl.ds(c * L, L)] = zeros16
            msg2[1, r, pl.ds(c * L, L)] = zeros16

    @pl.loop(0, DCH)
    def _(j):
        for c in range(128 // L):
            wd1[j, pl.ds(c * L, L)] = zeros16

    pend = []
    for k in range(RPT // (2 * WS)):
        for b in range(2):
            pend.append(pltpu.async_copy(
                msg2.at[b], acc_sh.at[pl.ds(sid * RPT + (2 * k + b) * WS, WS)], dsem))
    for k in range(RPT // 128):
        pend.append(pltpu.async_copy(
            wd1.at[0], deg1_sh.at[pl.ds(sid * RPT + k * 128, 128)], dsem))
        pend.append(pltpu.async_copy(
            wd1.at[0], deg2_sh.at[pl.ds(sid * RPT + k * 128, 128)], dsem))
    for p in pend:
        p.wait()
    plsc.subcore_barrier()

    @pl.loop(0, DROWS_PT // DCH)
    def _(chunk):
        row0 = sid * DROWS_PT + chunk * DCH
        pltpu.sync_copy(dstE.at[pl.ds(row0, DCH)], dstd)
        pltpu.sync_copy(w1E.at[pl.ds(row0, DCH)], wd1)
        pltpu.sync_copy(w2E.at[pl.ds(row0, DCH)], wd2)
        pend2 = []
        for j in range(DCH):
            pend2.append(pltpu.async_copy(
                wd1.at[j], deg1_sh.at[dstd.at[j]], dsem, add=True))
            pend2.append(pltpu.async_copy(
                wd2.at[j], deg2_sh.at[dstd.at[j]], dsem, add=True))
        for p in pend2:
            p.wait()

    plsc.subcore_barrier()

    off = sid * RPT

    @pl.loop(0, RPT // 128)
    def _(k):
        pltpu.sync_copy(deg1_sh.at[pl.ds(off + k * 128, 128)], wd1.at[0])
        pltpu.sync_copy(deg2_sh.at[pl.ds(off + k * 128, 128)], wd2.at[0])
        for buf in (wd1, wd2):
            for g in range(128 // L):
                d = buf[0, pl.ds(g * L, L)]
                bits = plsc.bitcast(d, jnp.int32)
                y = plsc.bitcast(jnp.int32(MAGIC) - (bits >> 1), jnp.float32)
                for _ in range(3):
                    y = y * (1.5 - 0.5 * d * y * y)
                buf[0, pl.ds(g * L, L)] = y
        pltpu.sync_copy(wd1.at[0], deg1_sh.at[pl.ds(off + k * 128, 128)])
        pltpu.sync_copy(wd2.at[0], deg2_sh.at[pl.ds(off + k * 128, 128)])

    plsc.subcore_barrier()
    pltpu.sync_copy(deg1_sh.at[pl.ds(0, N)], dinv1_t)
    pltpu.sync_copy(deg2_sh.at[pl.ds(0, N)], dinv2_t)

    sbase = cid * (E32 // NC) + sid * SPT

    def issue_load(k, b):
        return pltpu.async_copy(exy.at[pl.ds(sbase + k, 1)], eb.at[b], esem[b])

    def wait_load(b):
        pltpu.make_async_copy(exy.at[pl.ds(sbase, 1)], eb.at[b], esem[b]).wait()

    def issue_gather(b):
        return pltpu.async_copy(h_hbm.at[eb.at[b, 0, 0]], rows2.at[b], gsem[b])

    def wait_gather(b):
        pltpu.make_async_copy(h_hbm.at[eb.at[b, 0, 0]], rows2.at[b], gsem[b]).wait()

    def issue_scatter(b):
        return pltpu.async_copy(msg2.at[b], acc_sh.at[db.at[b, 0]], ssem[b],
                                add=True)

    def wait_scatter(b):
        pltpu.make_async_copy(msg2.at[b], acc_sh.at[db.at[b, 0]], ssem[b]).wait()

    issue_load(0, 0)
    issue_load(1, 1)
    wait_load(0)
    wait_load(1)
    issue_gather(0)
    issue_gather(1)

    def step_half(k, b, first):
        if not first:
            wait_scatter(b)
        for g in range(WS // L):
            s16 = eb[b, 0, 0, pl.ds(g * L, L)]
            d16 = eb[b, 0, 1, pl.ds(g * L, L)]
            db[b, 0, pl.ds(g * L, L)] = d16
            w1g = plsc.bitcast(eb[b, 0, 2, pl.ds(g * L, L)], jnp.float32)
            w2g = plsc.bitcast(eb[b, 0, 3, pl.ds(g * L, L)], jnp.float32)
            n1 = (plsc.load_gather(dinv1_t, [s16]) * w1g
                  * plsc.load_gather(dinv1_t, [d16]))
            n2 = (plsc.load_gather(dinv2_t, [s16]) * w2g
                  * plsc.load_gather(dinv2_t, [d16]))
            nb[b, 0, 0, pl.ds(g * L, L)] = n1
            nb[b, 0, 1, pl.ds(g * L, L)] = n2
        issue_load(k + 2, b)
        wait_gather(b)

        @pl.loop(0, WS, step=2)
        def _(e):
            for u in range(2):
                eidx = jnp.full((L,), e + u, jnp.int32)
                bn1 = plsc.load_gather(nb.at[b, 0, 0], [eidx])
                bn2 = plsc.load_gather(nb.at[b, 0, 1], [eidx])
                for c in range(D // L):
                    r1 = rows2[b, e + u, pl.ds(c * L, L)]
                    r2 = rows2[b, e + u, pl.ds(D + c * L, L)]
                    msg2[b, e + u, pl.ds(c * L, L)] = bn1 * r1 + bn2 * r2
        issue_scatter(b)
        wait_load(b)
        issue_gather(b)

    step_half(0, 0, True)
    step_half(1, 1, True)

    @pl.loop(2, SPT - 2, step=2)
    def _(k):
        step_half(k, 0, False)
        step_half(k + 1, 1, False)

    step_half(SPT - 2, 0, False)
    step_half(SPT - 1, 1, False)
    wait_gather(0)
    wait_gather(1)
    wait_scatter(0)
    wait_scatter(1)

    plsc.subcore_barrier()
    obase = cid * NPAD + sid * RPT
    pltpu.sync_copy(acc_sh.at[pl.ds(sid * RPT, RPT)], acc_hbm.at[pl.ds(obase, RPT)])


def _mm_body(x_ref, we_ref, wc_ref, wl_ref, h_ref):
    wl = wl_ref[...]
    m = jnp.concatenate(
        [jnp.dot(we_ref[...], wl[:D], preferred_element_type=jnp.float32),
         jnp.dot(wc_ref[...], wl[D:], preferred_element_type=jnp.float32)],
        axis=1)
    h_ref[...] = jnp.dot(x_ref[...], m, preferred_element_type=jnp.float32)


def _fin_body(a0_ref, a1_ref, be_ref, bc_ref, wl_ref, bl_ref, o_ref):
    wl = wl_ref[...]
    crow = (jnp.dot(be_ref[...], wl[:D], preferred_element_type=jnp.float32)
            + jnp.dot(bc_ref[...], wl[D:], preferred_element_type=jnp.float32)
            + bl_ref[...])
    o_ref[...] = a0_ref[0] + a1_ref[0] + crow


def kernel(x, edge_index, edge_attr, W_elec, b_elec, W_chem, b_chem, W_lin, b_lin):
    src = edge_index[0].astype(jnp.int32)
    dst = edge_index[1].astype(jnp.int32)
    loop = jnp.arange(N, dtype=jnp.int32)
    padi = jnp.zeros((E_PAD - E_TOT,), jnp.int32)
    padf = jnp.zeros((E_PAD - E_TOT,), jnp.float32)
    two = jnp.full((N,), 2.0, jnp.float32)
    srcp = jnp.concatenate([src, loop, padi])
    dstp = jnp.concatenate([dst, loop, padi])
    w1p = jnp.concatenate([edge_attr[:, 0], two, padf])
    w2p = jnp.concatenate([edge_attr[:, 1], two, padf])
    exy = jnp.concatenate(
        [jnp.stack([srcp.reshape(E32, WS), dstp.reshape(E32, WS),
                    lax.bitcast_convert_type(w1p, jnp.int32).reshape(E32, WS),
                    lax.bitcast_convert_type(w2p, jnp.int32).reshape(E32, WS)],
                   axis=1),
         jnp.zeros((2, 4, WS), jnp.int32)])
    dstE = dstp.reshape(E128, 128)
    w1E = w1p.reshape(E128, 128)
    w2E = w2p.reshape(E128, 128)

    nblk = 10
    h = pl.pallas_call(
        _mm_body,
        grid=(nblk,),
        in_specs=[
            pl.BlockSpec((N // nblk, D), lambda i: (i, 0)),
            pl.BlockSpec((D, D), lambda i: (0, 0)),
            pl.BlockSpec((D, D), lambda i: (0, 0)),
            pl.BlockSpec((2 * D, D), lambda i: (0, 0)),
        ],
        out_specs=pl.BlockSpec((N // nblk, DH2), lambda i: (i, 0)),
        out_shape=jax.ShapeDtypeStruct((N, DH2), jnp.float32),
    )(x, W_elec, W_chem, W_lin)

    mesh = plsc.VectorSubcoreMesh(core_axis_name="c", subcore_axis_name="s",
                                  num_cores=NC, num_subcores=NS)
    sc_fn = pl.kernel(
        _sc_body,
        out_type=jax.ShapeDtypeStruct((NC * NPAD, D), jnp.float32),
        mesh=mesh,
        compiler_params=pltpu.CompilerParams(use_tc_tiling_on_sc=False,
                                             needs_layout_passes=False),
        scratch_types=[
            pltpu.VMEM((N,), jnp.float32),
            pltpu.VMEM((N,), jnp.float32),
            pltpu.VMEM((2, WS, DH2), jnp.float32),
            pltpu.VMEM((2, WS, D), jnp.float32),
            pltpu.VMEM((2, 1, 4, WS), jnp.int32),
            pltpu.VMEM((2, 1, WS), jnp.int32),
            pltpu.VMEM((2, 1, 2, WS), jnp.float32),
            pltpu.VMEM((DCH, 128), jnp.int32),
            pltpu.VMEM((DCH, 128), jnp.float32),
            pltpu.VMEM((DCH, 128), jnp.float32),
            pltpu.SemaphoreType.DMA,
            pltpu.SemaphoreType.DMA,
            pltpu.SemaphoreType.DMA,
            pltpu.SemaphoreType.DMA,
            pltpu.SemaphoreType.DMA,
            pltpu.SemaphoreType.DMA,
            pltpu.SemaphoreType.DMA,
            pltpu.VMEM_SHARED((NPAD,), jnp.float32),
            pltpu.VMEM_SHARED((NPAD,), jnp.float32),
            pltpu.VMEM_SHARED((NPAD, D), jnp.float32),
        ],
    )
    acc = sc_fn(exy, dstE, w1E, w2E, h)
    acc3d = acc.reshape(NC, NPAD, D)

    out = pl.pallas_call(
        _fin_body,
        grid=(nblk,),
        in_specs=[
            pl.BlockSpec((1, N // nblk, D), lambda i: (0, i, 0)),
            pl.BlockSpec((1, N // nblk, D), lambda i: (1, i, 0)),
            pl.BlockSpec((1, D), lambda i: (0, 0)),
            pl.BlockSpec((1, D), lambda i: (0, 0)),
            pl.BlockSpec((2 * D, D), lambda i: (0, 0)),
            pl.BlockSpec((1, D), lambda i: (0, 0)),
        ],
        out_specs=pl.BlockSpec((N // nblk, D), lambda i: (i, 0)),
        out_shape=jax.ShapeDtypeStruct((N, D), jnp.float32),
    )(acc3d, acc3d, b_elec.reshape(1, D), b_chem.reshape(1, D),
      W_lin, b_lin.reshape(1, D))
    return out


# speedup vs baseline: 17.1954x; 1.0100x over previous
import jax
import jax.numpy as jnp
from jax import lax
from jax.experimental import pallas as pl
from jax.experimental.pallas import tpu as pltpu
from jax.experimental.pallas import tpu_sc as plsc

N = 10000
D = 128
DH2 = 256
E_IN = 320000
E_TOT = E_IN + N

NC, NS, L = 2, 16, 16
NTILES = NC * NS

WS = 32
SPT = 2 * (-(-E_TOT // (NTILES * WS * 2)))
E_PAD = NTILES * SPT * WS
E32 = E_PAD // WS
E128 = E_PAD // 128
DROWS_PT = E128 // NS
DCH = 3
NDCH = DROWS_PT // DCH
assert SPT % 2 == 0 and E128 * 128 == E_PAD
assert DROWS_PT * NS == E128 and DROWS_PT % DCH == 0 and NDCH % 2 == 0

NPAD = 10240
RPT = NPAD // NS
MAGIC = 0x5F3759DF


def _sc_body(exy, dstE, w1E, w2E, h_hbm, acc_hbm,
             dinv1_t, dinv2_t, rows2, msg2, eb, db, nb,
             dstd, wd1, wd2,
             gsem0, gsem1, ssem0, ssem1, esem0, esem1, dsem,
             deg1_sh, deg2_sh, acc_sh):
    cid = lax.axis_index("c")
    sid = lax.axis_index("s")
    zeros16 = jnp.zeros((L,), jnp.float32)
    gsem = (gsem0, gsem1)
    ssem = (ssem0, ssem1)
    esem = (esem0, esem1)

    @pl.loop(0, WS)
    def _(r):
        for c in range(D // L):
            msg2[0, r, pl.ds(c * L, L)] = zeros16
            msg2[1, r, pl.ds(c * L, L)] = zeros16

    @pl.loop(0, 1)
    def _(_j):
        for c in range(128 // L):
            wd1[0, 0, pl.ds(c * L, L)] = zeros16

    pend = []
    for k in range(RPT // (2 * WS)):
        for b in range(2):
            pend.append(pltpu.async_copy(
                msg2.at[b], acc_sh.at[pl.ds(sid * RPT + (2 * k + b) * WS, WS)], dsem))
    for k in range(RPT // 128):
        pend.append(pltpu.async_copy(
            wd1.at[0, 0], deg1_sh.at[pl.ds(sid * RPT + k * 128, 128)], dsem))
        pend.append(pltpu.async_copy(
            wd1.at[0, 0], deg2_sh.at[pl.ds(sid * RPT + k * 128, 128)], dsem))
    for p in pend:
        p.wait()
    plsc.subcore_barrier()

    def dload(c, p):
        row0 = sid * DROWS_PT + c * DCH
        pltpu.async_copy(dstE.at[pl.ds(row0, DCH)], dstd.at[p], esem[p])
        pltpu.async_copy(w1E.at[pl.ds(row0, DCH)], wd1.at[p], esem[p])
        pltpu.async_copy(w2E.at[pl.ds(row0, DCH)], wd2.at[p], esem[p])

    def dwait(p):
        pltpu.make_async_copy(dstE.at[pl.ds(0, DCH)], dstd.at[p], esem[p]).wait()
        pltpu.make_async_copy(w1E.at[pl.ds(0, DCH)], wd1.at[p], esem[p]).wait()
        pltpu.make_async_copy(w2E.at[pl.ds(0, DCH)], wd2.at[p], esem[p]).wait()

    def dchunk(c, p):
        dwait(p)
        pend2 = []
        for j in range(DCH):
            pend2.append(pltpu.async_copy(
                wd1.at[p, j], deg1_sh.at[dstd.at[p, j]], dsem, add=True))
            pend2.append(pltpu.async_copy(
                wd2.at[p, j], deg2_sh.at[dstd.at[p, j]], dsem, add=True))
        for q in pend2:
            q.wait()
        dload(c + 2, p)

    dload(0, 0)
    dload(1, 1)

    @pl.loop(0, NDCH, step=2)
    def _(c):
        dchunk(c, 0)
        dchunk(c + 1, 1)

    dwait(0)
    dwait(1)
    plsc.subcore_barrier()

    off = sid * RPT

    @pl.loop(0, RPT // 128)
    def _(k):
        pltpu.sync_copy(deg1_sh.at[pl.ds(off + k * 128, 128)], wd1.at[0, 0])
        pltpu.sync_copy(deg2_sh.at[pl.ds(off + k * 128, 128)], wd2.at[0, 0])
        for buf in (wd1, wd2):
            for g in range(128 // L):
                d = buf[0, 0, pl.ds(g * L, L)]
                bits = plsc.bitcast(d, jnp.int32)
                y = plsc.bitcast(jnp.int32(MAGIC) - (bits >> 1), jnp.float32)
                for _ in range(3):
                    y = y * (1.5 - 0.5 * d * y * y)
                buf[0, 0, pl.ds(g * L, L)] = y
        pltpu.sync_copy(wd1.at[0, 0], deg1_sh.at[pl.ds(off + k * 128, 128)])
        pltpu.sync_copy(wd2.at[0, 0], deg2_sh.at[pl.ds(off + k * 128, 128)])

    plsc.subcore_barrier()
    pltpu.sync_copy(deg1_sh.at[pl.ds(0, N)], dinv1_t)
    pltpu.sync_copy(deg2_sh.at[pl.ds(0, N)], dinv2_t)

    sbase = cid * (E32 // NC) + sid * SPT

    def issue_load(k, b):
        return pltpu.async_copy(exy.at[pl.ds(sbase + k, 1)], eb.at[b], esem[b])

    def wait_load(b):
        pltpu.make_async_copy(exy.at[pl.ds(sbase, 1)], eb.at[b], esem[b]).wait()

    def issue_gather(b):
        return pltpu.async_copy(h_hbm.at[eb.at[b, 0, 0]], rows2.at[b], gsem[b])

    def wait_gather(b):
        pltpu.make_async_copy(h_hbm.at[eb.at[b, 0, 0]], rows2.at[b], gsem[b]).wait()

    def issue_scatter(b):
        return pltpu.async_copy(msg2.at[b], acc_sh.at[db.at[b, 0]], ssem[b],
                                add=True)

    def wait_scatter(b):
        pltpu.make_async_copy(msg2.at[b], acc_sh.at[db.at[b, 0]], ssem[b]).wait()

    issue_load(0, 0)
    issue_load(1, 1)
    wait_load(0)
    wait_load(1)
    issue_gather(0)
    issue_gather(1)

    def step_half(k, b, first):
        if not first:
            wait_scatter(b)
        for g in range(WS // L):
            s16 = eb[b, 0, 0, pl.ds(g * L, L)]
            d16 = eb[b, 0, 1, pl.ds(g * L, L)]
            db[b, 0, pl.ds(g * L, L)] = d16
            w1g = plsc.bitcast(eb[b, 0, 2, pl.ds(g * L, L)], jnp.float32)
            w2g = plsc.bitcast(eb[b, 0, 3, pl.ds(g * L, L)], jnp.float32)
            n1 = (plsc.load_gather(dinv1_t, [s16]) * w1g
                  * plsc.load_gather(dinv1_t, [d16]))
            n2 = (plsc.load_gather(dinv2_t, [s16]) * w2g
                  * plsc.load_gather(dinv2_t, [d16]))
            nb[b, 0, 0, pl.ds(g * L, L)] = n1
            nb[b, 0, 1, pl.ds(g * L, L)] = n2
        issue_load(k + 2, b)
        wait_gather(b)

        @pl.loop(0, WS, step=2)
        def _(e):
            for u in range(2):
                eidx = jnp.full((L,), e + u, jnp.int32)
                bn1 = plsc.load_gather(nb.at[b, 0, 0], [eidx])
                bn2 = plsc.load_gather(nb.at[b, 0, 1], [eidx])
                for c in range(D // L):
                    r1 = rows2[b, e + u, pl.ds(c * L, L)]
                    r2 = rows2[b, e + u, pl.ds(D + c * L, L)]
                    msg2[b, e + u, pl.ds(c * L, L)] = bn1 * r1 + bn2 * r2
        issue_scatter(b)
        wait_load(b)
        issue_gather(b)

    step_half(0, 0, True)
    step_half(1, 1, True)

    @pl.loop(2, SPT - 2, step=2)
    def _(k):
        step_half(k, 0, False)
        step_half(k + 1, 1, False)

    step_half(SPT - 2, 0, False)
    step_half(SPT - 1, 1, False)
    wait_gather(0)
    wait_gather(1)
    wait_scatter(0)
    wait_scatter(1)

    plsc.subcore_barrier()
    obase = cid * NPAD + sid * RPT
    pltpu.sync_copy(acc_sh.at[pl.ds(sid * RPT, RPT)], acc_hbm.at[pl.ds(obase, RPT)])


def _mm_body(x_ref, we_ref, wc_ref, wl_ref, h_ref):
    wl = wl_ref[...]
    m = jnp.concatenate(
        [jnp.dot(we_ref[...], wl[:D], preferred_element_type=jnp.float32),
         jnp.dot(wc_ref[...], wl[D:], preferred_element_type=jnp.float32)],
        axis=1)
    h_ref[...] = jnp.dot(x_ref[...], m, preferred_element_type=jnp.float32)


def _fin_body(a0_ref, a1_ref, be_ref, bc_ref, wl_ref, bl_ref, o_ref):
    wl = wl_ref[...]
    crow = (jnp.dot(be_ref[...], wl[:D], preferred_element_type=jnp.float32)
            + jnp.dot(bc_ref[...], wl[D:], preferred_element_type=jnp.float32)
            + bl_ref[...])
    o_ref[...] = a0_ref[0] + a1_ref[0] + crow


def kernel(x, edge_index, edge_attr, W_elec, b_elec, W_chem, b_chem, W_lin, b_lin):
    src = edge_index[0].astype(jnp.int32)
    dst = edge_index[1].astype(jnp.int32)
    loop = jnp.arange(N, dtype=jnp.int32)
    padi = jnp.zeros((E_PAD - E_TOT,), jnp.int32)
    padf = jnp.zeros((E_PAD - E_TOT,), jnp.float32)
    two = jnp.full((N,), 2.0, jnp.float32)
    srcp = jnp.concatenate([src, loop, padi])
    dstp = jnp.concatenate([dst, loop, padi])
    w1p = jnp.concatenate([edge_attr[:, 0], two, padf])
    w2p = jnp.concatenate([edge_attr[:, 1], two, padf])
    exy = jnp.concatenate(
        [jnp.stack([srcp.reshape(E32, WS), dstp.reshape(E32, WS),
                    lax.bitcast_convert_type(w1p, jnp.int32).reshape(E32, WS),
                    lax.bitcast_convert_type(w2p, jnp.int32).reshape(E32, WS)],
                   axis=1),
         jnp.zeros((2, 4, WS), jnp.int32)])
    zpadi = jnp.zeros((2 * DCH, 128), jnp.int32)
    zpadf = jnp.zeros((2 * DCH, 128), jnp.float32)
    dstE = jnp.concatenate([dstp.reshape(E128, 128), zpadi])
    w1E = jnp.concatenate([w1p.reshape(E128, 128), zpadf])
    w2E = jnp.concatenate([w2p.reshape(E128, 128), zpadf])

    nblk = 10
    h = pl.pallas_call(
        _mm_body,
        grid=(nblk,),
        in_specs=[
            pl.BlockSpec((N // nblk, D), lambda i: (i, 0)),
            pl.BlockSpec((D, D), lambda i: (0, 0)),
            pl.BlockSpec((D, D), lambda i: (0, 0)),
            pl.BlockSpec((2 * D, D), lambda i: (0, 0)),
        ],
        out_specs=pl.BlockSpec((N // nblk, DH2), lambda i: (i, 0)),
        out_shape=jax.ShapeDtypeStruct((N, DH2), jnp.float32),
    )(x, W_elec, W_chem, W_lin)

    mesh = plsc.VectorSubcoreMesh(core_axis_name="c", subcore_axis_name="s",
                                  num_cores=NC, num_subcores=NS)
    sc_fn = pl.kernel(
        _sc_body,
        out_type=jax.ShapeDtypeStruct((NC * NPAD, D), jnp.float32),
        mesh=mesh,
        compiler_params=pltpu.CompilerParams(use_tc_tiling_on_sc=False,
                                             needs_layout_passes=False),
        scratch_types=[
            pltpu.VMEM((N,), jnp.float32),
            pltpu.VMEM((N,), jnp.float32),
            pltpu.VMEM((2, WS, DH2), jnp.float32),
            pltpu.VMEM((2, WS, D), jnp.float32),
            pltpu.VMEM((2, 1, 4, WS), jnp.int32),
            pltpu.VMEM((2, 1, WS), jnp.int32),
            pltpu.VMEM((2, 1, 2, WS), jnp.float32),
            pltpu.VMEM((2, DCH, 128), jnp.int32),
            pltpu.VMEM((2, DCH, 128), jnp.float32),
            pltpu.VMEM((2, DCH, 128), jnp.float32),
            pltpu.SemaphoreType.DMA,
            pltpu.SemaphoreType.DMA,
            pltpu.SemaphoreType.DMA,
            pltpu.SemaphoreType.DMA,
            pltpu.SemaphoreType.DMA,
            pltpu.SemaphoreType.DMA,
            pltpu.SemaphoreType.DMA,
            pltpu.VMEM_SHARED((NPAD,), jnp.float32),
            pltpu.VMEM_SHARED((NPAD,), jnp.float32),
            pltpu.VMEM_SHARED((NPAD, D), jnp.float32),
        ],
    )
    acc = sc_fn(exy, dstE, w1E, w2E, h)
    acc3d = acc.reshape(NC, NPAD, D)

    out = pl.pallas_call(
        _fin_body,
        grid=(nblk,),
        in_specs=[
            pl.BlockSpec((1, N // nblk, D), lambda i: (0, i, 0)),
            pl.BlockSpec((1, N // nblk, D), lambda i: (1, i, 0)),
            pl.BlockSpec((1, D), lambda i: (0, 0)),
            pl.BlockSpec((1, D), lambda i: (0, 0)),
            pl.BlockSpec((2 * D, D), lambda i: (0, 0)),
            pl.BlockSpec((1, D), lambda i: (0, 0)),
        ],
        out_specs=pl.BlockSpec((N // nblk, D), lambda i: (i, 0)),
        out_shape=jax.ShapeDtypeStruct((N, D), jnp.float32),
    )(acc3d, acc3d, b_elec.reshape(1, D), b_chem.reshape(1, D),
      W_lin, b_lin.reshape(1, D))
    return out
